# baseline (speedup 1.0000x reference)
import functools

import numpy as np
import jax
import jax.numpy as jnp
from jax import lax
from jax.experimental import pallas as pl
from jax.experimental.pallas import tpu as pltpu
from jax.experimental.pallas import tpu_sc as plsc

_TIMESTEPS = 1000
_BETAS = np.linspace(0.0001, 0.02, _TIMESTEPS, dtype=np.float64)
_AC = np.cumprod(1.0 - _BETAS)
_SQRT_AC = np.sqrt(_AC).astype(np.float32)
_SQRT_OMAC = np.sqrt(1.0 - _AC).astype(np.float32)

_BATCH = 64
_LANES = 256
_ROWS_PER_BATCH = 3 * 256
_BATCHES_PER_BLOCK = 4
_BLOCK_ROWS = _ROWS_PER_BATCH * _BATCHES_PER_BLOCK


def _sc_gather_body(t_hbm, a_tab_hbm, b_tab_hbm, a_out_hbm, b_out_hbm,
                    idx_v, a_v, b_v, sem):
    wid = lax.axis_index("s") * 2 + lax.axis_index("c")

    @pl.when(wid == 0)
    def _():
        pltpu.sync_copy(t_hbm, idx_v)
        pltpu.async_copy(a_tab_hbm.at[idx_v], a_v, sem).wait()
        pltpu.async_copy(b_tab_hbm.at[idx_v], b_v, sem).wait()
        pltpu.sync_copy(a_v, a_out_hbm)
        pltpu.sync_copy(b_v, b_out_hbm)


def _sc_gather(t, a_tab, b_tab):
    mesh = plsc.VectorSubcoreMesh(core_axis_name="c", subcore_axis_name="s")
    fn = functools.partial(
        pl.kernel,
        out_type=(
            jax.ShapeDtypeStruct((_BATCH,), jnp.float32),
            jax.ShapeDtypeStruct((_BATCH,), jnp.float32),
        ),
        mesh=mesh,
        scratch_types=[
            pltpu.VMEM((_BATCH,), jnp.int32),
            pltpu.VMEM((_BATCH,), jnp.float32),
            pltpu.VMEM((_BATCH,), jnp.float32),
            pltpu.SemaphoreType.DMA,
        ],
    )(_sc_gather_body)
    return fn(t, a_tab, b_tab)


def _fma_body(a_ref, b_ref, x_ref, n_ref, o_ref):
    blk = pl.program_id(0)
    for j in range(_BATCHES_PER_BLOCK):
        bidx = blk * _BATCHES_PER_BLOCK + j
        a = a_ref[bidx]
        b = b_ref[bidx]
        sl = pl.ds(j * _ROWS_PER_BATCH, _ROWS_PER_BATCH)
        o_ref[sl, :] = a * x_ref[sl, :] + b * n_ref[sl, :]


def kernel(x_start, t, noise):
    batch = x_start.shape[0]
    rows = batch * _ROWS_PER_BATCH
    a_vec, b_vec = _sc_gather(
        t.astype(jnp.int32), jnp.asarray(_SQRT_AC), jnp.asarray(_SQRT_OMAC))
    x2 = jnp.transpose(x_start, (0, 3, 1, 2)).reshape(rows, _LANES)
    n2 = jnp.transpose(noise, (0, 3, 1, 2)).reshape(rows, _LANES)
    grid = (rows // _BLOCK_ROWS,)
    out = pl.pallas_call(
        _fma_body,
        grid=grid,
        in_specs=[
            pl.BlockSpec(memory_space=pltpu.SMEM),
            pl.BlockSpec(memory_space=pltpu.SMEM),
            pl.BlockSpec((_BLOCK_ROWS, _LANES), lambda i: (i, 0)),
            pl.BlockSpec((_BLOCK_ROWS, _LANES), lambda i: (i, 0)),
        ],
        out_specs=pl.BlockSpec((_BLOCK_ROWS, _LANES), lambda i: (i, 0)),
        out_shape=jax.ShapeDtypeStruct((rows, _LANES), jnp.float32),
    )(a_vec, b_vec, x2, n2)
    out = out.reshape(batch, 3, 256, 256)
    return jnp.transpose(out, (0, 2, 3, 1))


# speedup vs baseline: 1.4424x; 1.4424x over previous
import numpy as np
import jax
import jax.numpy as jnp
from jax.experimental import pallas as pl
from jax.experimental.pallas import tpu as pltpu

_TIMESTEPS = 1000
_BETAS = np.linspace(0.0001, 0.02, _TIMESTEPS, dtype=np.float64)
_AC = np.cumprod(1.0 - _BETAS)
_TABLES = np.stack([np.sqrt(_AC), np.sqrt(1.0 - _AC)]).astype(np.float32)

_LANES = 256
_ROWS_PER_BATCH = 3 * 256
_BATCHES_PER_BLOCK = 4
_BLOCK_ROWS = _ROWS_PER_BATCH * _BATCHES_PER_BLOCK


def _fma_body(t_ref, tab_ref, x_ref, n_ref, o_ref):
    blk = pl.program_id(0)
    for j in range(_BATCHES_PER_BLOCK):
        bidx = blk * _BATCHES_PER_BLOCK + j
        tt = t_ref[bidx]
        a = tab_ref[0, tt]
        b = tab_ref[1, tt]
        sl = pl.ds(j * _ROWS_PER_BATCH, _ROWS_PER_BATCH)
        o_ref[sl, :] = a * x_ref[sl, :] + b * n_ref[sl, :]


def kernel(x_start, t, noise):
    batch = x_start.shape[0]
    rows = batch * _ROWS_PER_BATCH
    x2 = jnp.transpose(x_start, (0, 3, 1, 2)).reshape(rows, _LANES)
    n2 = jnp.transpose(noise, (0, 3, 1, 2)).reshape(rows, _LANES)
    grid = (rows // _BLOCK_ROWS,)
    out = pl.pallas_call(
        _fma_body,
        grid=grid,
        in_specs=[
            pl.BlockSpec(memory_space=pltpu.SMEM),
            pl.BlockSpec(memory_space=pltpu.SMEM),
            pl.BlockSpec((_BLOCK_ROWS, _LANES), lambda i: (i, 0)),
            pl.BlockSpec((_BLOCK_ROWS, _LANES), lambda i: (i, 0)),
        ],
        out_specs=pl.BlockSpec((_BLOCK_ROWS, _LANES), lambda i: (i, 0)),
        out_shape=jax.ShapeDtypeStruct((rows, _LANES), jnp.float32),
    )(t.astype(jnp.int32), jnp.asarray(_TABLES), x2, n2)
    out = out.reshape(batch, 3, 256, 256)
    return jnp.transpose(out, (0, 2, 3, 1))
